# baseline (speedup 1.0000x reference)
import functools

import jax
import jax.numpy as jnp
from jax import lax
from jax.experimental import pallas as pl
from jax.experimental.pallas import tpu as pltpu
from jax.experimental.pallas import tpu_sc as plsc

_N_EDGE_ROWS = 320000
_D_EDGE = 16
_N_WORKERS = 32
_HEAD_ROWS = 153600
_ROWS_PER_WORKER = _HEAD_ROWS // _N_WORKERS
_CHUNK = 400
_N_CHUNKS = _ROWS_PER_WORKER // _CHUNK

_TAIL_ROWS = _N_EDGE_ROWS - _HEAD_ROWS
_GRID = 25
_X_ROWS = 10000 // _GRID
_E_ROWS = _TAIL_ROWS // _GRID


def _tc_copy_body(x_ref, e_ref, u_ref, xo_ref, eo_ref, uo_ref):
    xo_ref[...] = x_ref[...]
    eo_ref[...] = e_ref[...]
    uo_ref[...] = u_ref[...]


def _tc_copy(x, e_tail, u):
    return pl.pallas_call(
        _tc_copy_body,
        grid=(_GRID,),
        out_shape=(
            jax.ShapeDtypeStruct(x.shape, x.dtype),
            jax.ShapeDtypeStruct(e_tail.shape, e_tail.dtype),
            jax.ShapeDtypeStruct(u.shape, u.dtype),
        ),
        in_specs=[
            pl.BlockSpec((_X_ROWS, 128), lambda i: (i, 0)),
            pl.BlockSpec((_E_ROWS, _D_EDGE), lambda i: (i, 0)),
            pl.BlockSpec((64, 64), lambda i: (0, 0)),
        ],
        out_specs=(
            pl.BlockSpec((_X_ROWS, 128), lambda i: (i, 0)),
            pl.BlockSpec((_E_ROWS, _D_EDGE), lambda i: (i, 0)),
            pl.BlockSpec((64, 64), lambda i: (0, 0)),
        ),
    )(x, e_tail, u)


@functools.partial(
    pl.kernel,
    mesh=plsc.VectorSubcoreMesh(core_axis_name="c", subcore_axis_name="s"),
    out_type=jax.ShapeDtypeStruct((_HEAD_ROWS, _D_EDGE), jnp.float32),
    scratch_types=[
        pltpu.VMEM((_CHUNK, _D_EDGE), jnp.float32),
        pltpu.VMEM((_CHUNK, _D_EDGE), jnp.float32),
        pltpu.SemaphoreType.DMA,
        pltpu.SemaphoreType.DMA,
    ],
)
def _sc_copy(e_hbm, out_hbm, buf0, buf1, sem0, sem1):
    wid = lax.axis_index("s") * 2 + lax.axis_index("c")
    base = wid * _ROWS_PER_WORKER
    bufs = (buf0, buf1)
    sems = (sem0, sem1)

    def _start_fetch(i):
        c = pltpu.make_async_copy(
            e_hbm.at[pl.ds(base + i * _CHUNK, _CHUNK)], bufs[i % 2], sems[i % 2]
        )
        c.start()
        return c

    pending = _start_fetch(0)
    for i in range(_N_CHUNKS):
        pending.wait()
        if i + 1 < _N_CHUNKS:
            nxt = _start_fetch(i + 1)
        pltpu.sync_copy(bufs[i % 2], out_hbm.at[pl.ds(base + i * _CHUNK, _CHUNK)])
        if i + 1 < _N_CHUNKS:
            pending = nxt


def kernel(x, edge_index, edge_attr, u, batch):
    del edge_index, batch
    eo_head = _sc_copy(edge_attr[:_HEAD_ROWS])
    xo, eo_tail, uo = _tc_copy(x, edge_attr[_HEAD_ROWS:], u)
    eo = jnp.concatenate([eo_head, eo_tail], axis=0)
    return xo, eo, uo


# speedup vs baseline: 1.8094x; 1.8094x over previous
import jax
from jax.experimental import pallas as pl

_GRID = 10
_X_ROWS = 10000 // _GRID


def _copy_body(x_ref, e_ref, u_ref, xo_ref, eo_ref, uo_ref):
    del e_ref, eo_ref
    xo_ref[...] = x_ref[...]
    uo_ref[...] = u_ref[...]


def kernel(x, edge_index, edge_attr, u, batch):
    del edge_index, batch
    return pl.pallas_call(
        _copy_body,
        grid=(_GRID,),
        out_shape=(
            jax.ShapeDtypeStruct(x.shape, x.dtype),
            jax.ShapeDtypeStruct(edge_attr.shape, edge_attr.dtype),
            jax.ShapeDtypeStruct(u.shape, u.dtype),
        ),
        in_specs=[
            pl.BlockSpec((_X_ROWS, 128), lambda i: (i, 0)),
            pl.BlockSpec(memory_space=pl.ANY),
            pl.BlockSpec((64, 64), lambda i: (0, 0)),
        ],
        out_specs=(
            pl.BlockSpec((_X_ROWS, 128), lambda i: (i, 0)),
            pl.BlockSpec(memory_space=pl.ANY),
            pl.BlockSpec((64, 64), lambda i: (0, 0)),
        ),
        input_output_aliases={1: 1},
    )(x, edge_attr, u)


# speedup vs baseline: 7.3593x; 4.0672x over previous
import functools

import jax
import jax.numpy as jnp
from jax import lax
from jax.experimental import pallas as pl
from jax.experimental.pallas import tpu as pltpu
from jax.experimental.pallas import tpu_sc as plsc

_N_X_ROWS = 10000
_D_FEAT = 128
_N_WORKERS = 32
_CHUNK = 40
_N_CHUNKS = _N_X_ROWS // _CHUNK
_MAX_K = -(-_N_CHUNKS // _N_WORKERS)


@functools.partial(
    pl.kernel,
    mesh=plsc.VectorSubcoreMesh(core_axis_name="c", subcore_axis_name="s"),
    out_type=jax.ShapeDtypeStruct((_N_X_ROWS, _D_FEAT), jnp.float32),
    scratch_types=[
        pltpu.VMEM((_CHUNK, _D_FEAT), jnp.float32),
        pltpu.VMEM((_CHUNK, _D_FEAT), jnp.float32),
        pltpu.SemaphoreType.DMA,
        pltpu.SemaphoreType.DMA,
    ],
)
def _sc_copy_x(x_hbm, xo_hbm, buf0, buf1, sem0, sem1):
    wid = lax.axis_index("s") * 2 + lax.axis_index("c")
    bufs = (buf0, buf1)
    sems = (sem0, sem1)

    def _copy_desc(k):
        c = wid + k * _N_WORKERS
        return pltpu.make_async_copy(
            x_hbm.at[pl.ds(c * _CHUNK, _CHUNK)], bufs[k % 2], sems[k % 2]
        )

    _copy_desc(0).start()
    for k in range(_MAX_K):
        c = wid + k * _N_WORKERS

        @pl.when(c < _N_CHUNKS)
        def _(k=k, c=c):
            _copy_desc(k).wait()
            if k + 1 < _MAX_K:

                @pl.when(wid + (k + 1) * _N_WORKERS < _N_CHUNKS)
                def _(k=k):
                    _copy_desc(k + 1).start()

            pltpu.sync_copy(bufs[k % 2], xo_hbm.at[pl.ds(c * _CHUNK, _CHUNK)])


def _tc_copy_u_body(u_ref, uo_ref):
    uo_ref[...] = u_ref[...]


def _tc_copy_u(u):
    return pl.pallas_call(
        _tc_copy_u_body,
        out_shape=jax.ShapeDtypeStruct(u.shape, u.dtype),
    )(u)


def kernel(x, edge_index, edge_attr, u, batch):
    del edge_index, batch
    xo = _sc_copy_x(x)
    uo = _tc_copy_u(u)
    return xo, edge_attr, uo


# speedup vs baseline: 7.3886x; 1.0040x over previous
import functools

import jax
import jax.numpy as jnp
from jax import lax
from jax.experimental import pallas as pl
from jax.experimental.pallas import tpu as pltpu
from jax.experimental.pallas import tpu_sc as plsc

_N_X_ROWS = 10000
_D_FEAT = 128
_N_WORKERS = 32
_CHUNK = 40
_N_CHUNKS = _N_X_ROWS // _CHUNK
_MAX_K = -(-_N_CHUNKS // _N_WORKERS)


@functools.partial(
    pl.kernel,
    mesh=plsc.VectorSubcoreMesh(core_axis_name="c", subcore_axis_name="s"),
    out_type=jax.ShapeDtypeStruct((_N_X_ROWS, _D_FEAT), jnp.float32),
    scratch_types=[
        pltpu.VMEM((_CHUNK, _D_FEAT), jnp.float32),
        pltpu.VMEM((_CHUNK, _D_FEAT), jnp.float32),
        pltpu.SemaphoreType.DMA,
        pltpu.SemaphoreType.DMA,
        pltpu.SemaphoreType.DMA,
        pltpu.SemaphoreType.DMA,
    ],
)
def _sc_copy_x(x_hbm, xo_hbm, buf0, buf1, isem0, isem1, osem0, osem1):
    wid = lax.axis_index("s") * 2 + lax.axis_index("c")
    bufs = (buf0, buf1)
    isems = (isem0, isem1)
    osems = (osem0, osem1)

    def _c(k):
        return wid + k * _N_WORKERS

    def _in_desc(k):
        return pltpu.make_async_copy(
            x_hbm.at[pl.ds(_c(k) * _CHUNK, _CHUNK)], bufs[k % 2], isems[k % 2]
        )

    def _out_desc(k):
        return pltpu.make_async_copy(
            bufs[k % 2], xo_hbm.at[pl.ds(_c(k) * _CHUNK, _CHUNK)], osems[k % 2]
        )

    def _guarded(pred, fn):
        pl.when(pred)(fn)

    _in_desc(0).start()
    for k in range(_MAX_K):
        _guarded(_c(k) < _N_CHUNKS, lambda k=k: _in_desc(k).wait())
        if k >= 1:
            _guarded(_c(k - 1) < _N_CHUNKS, lambda k=k: _out_desc(k - 1).wait())
        if k + 1 < _MAX_K:
            _guarded(_c(k + 1) < _N_CHUNKS, lambda k=k: _in_desc(k + 1).start())
        _guarded(_c(k) < _N_CHUNKS, lambda k=k: _out_desc(k).start())
    _guarded(_c(_MAX_K - 1) < _N_CHUNKS, lambda: _out_desc(_MAX_K - 1).wait())


def _tc_copy_u_body(u_ref, uo_ref):
    uo_ref[...] = u_ref[...]


def _tc_copy_u(u):
    return pl.pallas_call(
        _tc_copy_u_body,
        out_shape=jax.ShapeDtypeStruct(u.shape, u.dtype),
    )(u)


def kernel(x, edge_index, edge_attr, u, batch):
    del edge_index, batch
    xo = _sc_copy_x(x)
    uo = _tc_copy_u(u)
    return xo, edge_attr, uo
